# baseline (speedup 1.0000x reference)
import functools

import jax
import jax.numpy as jnp
from jax import lax
from jax.experimental import pallas as pl
from jax.experimental.pallas import tpu as pltpu
from jax.experimental.pallas import tpu_sc as plsc

_LANES = 16


@functools.cache
def _build_gather(rows, seq_len, dim, tokens_per_batch, rpw, num_workers):
    active = rows // rpw
    assert active * rpw == rows and active <= num_workers
    mesh = plsc.VectorSubcoreMesh(core_axis_name="c", subcore_axis_name="s")

    nch = 2
    cpw = rpw // nch
    assert cpw % 8 == 0

    @functools.partial(
        pl.kernel,
        mesh=mesh,
        out_type=jax.ShapeDtypeStruct((rows, dim), jnp.float32),
        scratch_types=[
            pltpu.VMEM((rpw,), jnp.int32),
            pltpu.VMEM((rpw, dim), jnp.float32),
            pltpu.SemaphoreType.DMA,
            pltpu.SemaphoreType.DMA,
            pltpu.SemaphoreType.DMA,
        ],
    )
    def gather_kernel(table_hbm, tok_hbm, out_hbm, idx_v, rows_v, gsem0, gsem1, wsem):
        wid = lax.axis_index("s") * 2 + lax.axis_index("c")

        @pl.when(wid < active)
        def _():
            base = wid * rpw
            pltpu.sync_copy(tok_hbm.at[pl.ds(base, rpw)], idx_v)
            gsems = [gsem0, gsem1]
            gathers = [
                pltpu.async_copy(
                    table_hbm.at[idx_v.at[pl.ds(ch * cpw, cpw)]],
                    rows_v.at[pl.ds(ch * cpw, cpw)],
                    gsems[ch],
                )
                for ch in range(nch)
            ]
            writes = []
            for ch in range(nch):
                gathers[ch].wait()
                writes.append(
                    pltpu.async_copy(
                        rows_v.at[pl.ds(ch * cpw, cpw)],
                        out_hbm.at[pl.ds(base + ch * cpw, cpw)],
                        wsem,
                    )
                )
            for w in writes:
                w.wait()

    return gather_kernel


def kernel(sequence_embedding, tokens):
    batch, seq_len, dim = sequence_embedding.shape
    _, tokens_per_batch = tokens.shape
    rows = batch * tokens_per_batch
    table = sequence_embedding.reshape(batch * seq_len, dim)
    offsets = jnp.arange(batch, dtype=tokens.dtype)[:, None] * seq_len
    flat_tokens = (tokens + offsets).reshape(rows)
    gather = _build_gather(rows, seq_len, dim, tokens_per_batch, 32, 32)
    out = gather(table, flat_tokens)
    return out.reshape(batch, tokens_per_batch * dim)
